# baseline (speedup 1.0000x reference)
import functools

import jax
import jax.numpy as jnp
from jax import lax
from jax.experimental import pallas as pl
from jax.experimental.pallas import tpu as pltpu
from jax.experimental.pallas import tpu_sc as plsc

_N = 10000
_F = 128
_NPAD = 10240
_PADIDX = _NPAD - 1
_CHUNK = 128
_NC = 2
_NS = 16
_NW = _NC * _NS
_RPS = _NPAD // _NS
_EPS = 1e-5
_TAU = 0.1


def _leaky(v):
    return jnp.where(v >= 0, v, 0.01 * v)


@functools.lru_cache(maxsize=None)
def _sc_scatter_kernel(cpw: int, with_degrees: bool):
    mesh = plsc.VectorSubcoreMesh(core_axis_name="c", subcore_axis_name="s")
    out_type = [jax.ShapeDtypeStruct((_NC, _NPAD, _F), jnp.float32)]
    if with_degrees:
        out_type += [jax.ShapeDtypeStruct((_NC, _NPAD), jnp.float32)] * 2
    scratch = [
        pltpu.VMEM((2, 2, _CHUNK), jnp.int32),
        pltpu.VMEM((2, _CHUNK, _F), jnp.float32),
        pltpu.VMEM_SHARED((_NPAD, _F), jnp.float32),
        pltpu.SemaphoreType.DMA,
        pltpu.SemaphoreType.DMA,
        pltpu.SemaphoreType.DMA,
        pltpu.SemaphoreType.DMA,
    ]
    if with_degrees:
        scratch += [
            pltpu.VMEM((_CHUNK,), jnp.float32),
            pltpu.VMEM_SHARED((_NPAD,), jnp.float32),
            pltpu.VMEM_SHARED((_NPAD,), jnp.float32),
        ]

    def body(*refs):
        if with_degrees:
            (src, idx, z2d, z1d,
             out, dout, bout,
             ibuf, rows, acc, sga, sgb, ssa, ssb, ones_v, dacc, bacc) = refs
        else:
            (src, idx, z2d,
             out, ibuf, rows, acc, sga, sgb, ssa, ssb) = refs
        cid = lax.axis_index("c")
        sid = lax.axis_index("s")
        w = cid * _NS + sid
        j0 = w * cpw
        r0 = sid * _RPS
        pltpu.sync_copy(z2d.at[pl.ds(r0, _RPS)], acc.at[pl.ds(r0, _RPS)])
        if with_degrees:
            pltpu.sync_copy(z1d.at[pl.ds(r0, _RPS)], dacc.at[pl.ds(r0, _RPS)])
            pltpu.sync_copy(z1d.at[pl.ds(r0, _RPS)], bacc.at[pl.ds(r0, _RPS)])
            for i in range(_CHUNK // 16):
                ones_v[pl.ds(i * 16, 16)] = jnp.ones((16,), jnp.float32)
        plsc.subcore_barrier()
        pltpu.sync_copy(idx.at[j0], ibuf.at[0])
        pltpu.async_copy(src.at[ibuf.at[0, 0]], rows.at[0], sga)
        pltpu.sync_copy(idx.at[j0 + 1], ibuf.at[1])
        pltpu.async_copy(src.at[ibuf.at[1, 0]], rows.at[1], sgb)

        def step(t, carry):
            for b, gsem, ssem in ((0, sga, ssa), (1, sgb, ssb)):
                pltpu.make_async_copy(src.at[ibuf.at[b, 0]], rows.at[b], gsem).wait()
                pltpu.async_copy(rows.at[b], acc.at[ibuf.at[b, 1]], ssem, add=True)
                if with_degrees:
                    pltpu.sync_copy(ones_v, dacc.at[ibuf.at[b, 0]], add=True)
                    pltpu.sync_copy(ones_v, bacc.at[ibuf.at[b, 1]], add=True)
            for b, gsem, ssem in ((0, sga, ssa), (1, sgb, ssb)):
                j = t * 2 + b
                pltpu.make_async_copy(rows.at[b], acc.at[ibuf.at[b, 1]], ssem).wait()

                @pl.when(t < cpw // 2 - 1)
                def _issue():
                    pltpu.sync_copy(idx.at[j0 + j + 2], ibuf.at[b])
                    pltpu.async_copy(src.at[ibuf.at[b, 0]], rows.at[b], gsem)
            return carry

        lax.fori_loop(0, cpw // 2, step, 0)
        plsc.subcore_barrier()
        pltpu.sync_copy(acc.at[pl.ds(r0, _RPS)], out.at[cid, pl.ds(r0, _RPS)])
        if with_degrees:
            pltpu.sync_copy(dacc.at[pl.ds(r0, _RPS)], dout.at[cid, pl.ds(r0, _RPS)])
            pltpu.sync_copy(bacc.at[pl.ds(r0, _RPS)], bout.at[cid, pl.ds(r0, _RPS)])

    return pl.kernel(
        body,
        out_type=tuple(out_type) if with_degrees else out_type[0],
        mesh=mesh,
        scratch_types=scratch,
    )


def _tc_matmul(xp, W):
    def body(x_ref, w_ref, o_ref):
        o_ref[...] = jnp.dot(x_ref[...], w_ref[...],
                             preferred_element_type=jnp.float32)
    return pl.pallas_call(
        body, out_shape=jax.ShapeDtypeStruct((_NPAD, _F), jnp.float32))(xp, W)


def _tc_combine_first(e_part, dcnt, bcnt):
    def body(e_ref, d_ref, b_ref, eo_ref, dinv_ref, binv_ref):
        mask = lax.broadcasted_iota(jnp.int32, (_NPAD, 1), 0) < _N
        dc = d_ref[0] + d_ref[1]
        bc = b_ref[0] + b_ref[1]
        dinv = jnp.where(mask & (dc > 0), 1.0 / dc, 0.0)
        binv = jnp.where(mask & (bc > 0), 1.0 / bc, 0.0)
        dinv_ref[...] = dinv
        binv_ref[...] = binv
        eo_ref[...] = (e_ref[0] + e_ref[1]) * binv
    return pl.pallas_call(body, out_shape=(
        jax.ShapeDtypeStruct((_NPAD, _F), jnp.float32),
        jax.ShapeDtypeStruct((_NPAD, 1), jnp.float32),
        jax.ShapeDtypeStruct((_NPAD, 1), jnp.float32),
    ))(e_part, dcnt, bcnt)


def _tc_combine(e_part, binv):
    def body(e_ref, bi_ref, eo_ref):
        eo_ref[...] = (e_ref[0] + e_ref[1]) * bi_ref[...]
    return pl.pallas_call(
        body, out_shape=jax.ShapeDtypeStruct((_NPAD, _F), jnp.float32))(e_part, binv)


def _tc_layer(o_part, dinv, bias, gw, gb, gms, W):
    def body(o_ref, di_ref, bi_ref, gw_ref, gb_ref, gms_ref, w_ref, out_ref):
        mask = lax.broadcasted_iota(jnp.int32, (_NPAD, 1), 0) < _N
        h = (o_ref[0] + o_ref[1]) * di_ref[...] + bi_ref[...]
        h = jnp.where(mask, h, 0.0)
        mean = jnp.sum(h, axis=0, keepdims=True) * (1.0 / _N)
        hc = jnp.where(mask, h - mean * gms_ref[...], 0.0)
        var = jnp.sum(hc * hc, axis=0, keepdims=True) * (1.0 / _N)
        g = hc * lax.rsqrt(var + _EPS) * gw_ref[...] + gb_ref[...]
        g = jnp.where(mask, _leaky(g), 0.0)
        out_ref[...] = jnp.dot(g, w_ref[...], preferred_element_type=jnp.float32)
    return pl.pallas_call(
        body, out_shape=jax.ShapeDtypeStruct((_NPAD, _F), jnp.float32))(
            o_part, dinv, bias, gw, gb, gms, W)


def _tc_head(o_part, dinv, bias, gw, gb, gms, mW1, mb1, mW2, mb2, gum):
    def body(o_ref, di_ref, bi_ref, gw_ref, gb_ref, gms_ref,
             w1_ref, b1_ref, w2_ref, b2_ref, g_ref, out_ref):
        mask = lax.broadcasted_iota(jnp.int32, (_NPAD, 1), 0) < _N
        h = (o_ref[0] + o_ref[1]) * di_ref[...] + bi_ref[...]
        h = jnp.where(mask, h, 0.0)
        mean = jnp.sum(h, axis=0, keepdims=True) * (1.0 / _N)
        hc = jnp.where(mask, h - mean * gms_ref[...], 0.0)
        var = jnp.sum(hc * hc, axis=0, keepdims=True) * (1.0 / _N)
        g = hc * lax.rsqrt(var + _EPS) * gw_ref[...] + gb_ref[...]
        g = jnp.where(mask, _leaky(g), 0.0)
        m = jnp.dot(g, w1_ref[...], preferred_element_type=jnp.float32) + b1_ref[...]
        m = jnp.where(mask, m, 0.0)
        mmean = jnp.sum(m, axis=0, keepdims=True) * (1.0 / _N)
        mc = jnp.where(mask, m - mmean, 0.0)
        mvar = jnp.sum(mc * mc, axis=0, keepdims=True) * (1.0 / _N)
        mi = _leaky(mc * lax.rsqrt(mvar + _EPS))
        logits = jnp.dot(mi, w2_ref[...], preferred_element_type=jnp.float32) + b2_ref[...]
        z = (logits + g_ref[...]) * (1.0 / _TAU)
        z = z - jnp.max(z, axis=1, keepdims=True)
        ez = jnp.exp(z)
        out_ref[...] = ez / jnp.sum(ez, axis=1, keepdims=True)
    return pl.pallas_call(
        body, out_shape=jax.ShapeDtypeStruct((_NPAD, 16), jnp.float32))(
            o_part, dinv, bias, gw, gb, gms, mW1, mb1, mW2, mb2, gum)


def kernel(x, inc_idx, W1, b1, gn1_w, gn1_b, gn1_ms, W2, b2, gn2_w, gn2_b,
           gn2_ms, mW1, mb1, mW2, mb2, gumbel):
    f32 = jnp.float32
    row = inc_idx[0].astype(jnp.int32)
    col = inc_idx[1].astype(jnp.int32)
    nnz = row.shape[0]
    chunks = -(-nnz // _CHUNK)
    cpw = -(-chunks // _NW)
    cpw += cpw % 2
    nnzp = cpw * _NW * _CHUNK
    pad = _N + jnp.arange(nnzp - nnz, dtype=jnp.int32) % (_NPAD - _N)
    rowp = jnp.concatenate([row, pad]).reshape(-1, _CHUNK)
    colp = jnp.concatenate([col, pad]).reshape(-1, _CHUNK)
    idx_ne = jnp.stack([rowp, colp], axis=1)
    idx_en = jnp.stack([colp, rowp], axis=1)
    xp = jnp.zeros((_NPAD, _F), f32).at[:_N, :].set(x)
    z2d = jnp.zeros((_NPAD, _F), f32)
    z1d = jnp.zeros((_NPAD,), f32)
    gum = jnp.zeros((_NPAD, 16), f32).at[:_N, :].set(gumbel)

    scat_deg = _sc_scatter_kernel(cpw, True)
    scat = _sc_scatter_kernel(cpw, False)

    x1 = _tc_matmul(xp, W1)
    e_p, dcnt, bcnt = scat_deg(x1, idx_ne, z2d, z1d)
    e_s, dinv, binv = _tc_combine_first(
        e_p, dcnt.reshape(_NC, _NPAD, 1), bcnt.reshape(_NC, _NPAD, 1))
    o_p = scat(e_s, idx_en, z2d)
    x2 = _tc_layer(o_p, dinv, b1.reshape(1, _F), gn1_w.reshape(1, _F),
                   gn1_b.reshape(1, _F), gn1_ms.reshape(1, _F), W2)
    e2_p = scat(x2, idx_ne, z2d)
    e2_s = _tc_combine(e2_p, binv)
    o2_p = scat(e2_s, idx_en, z2d)
    y = _tc_head(o2_p, dinv, b2.reshape(1, _F), gn2_w.reshape(1, _F),
                 gn2_b.reshape(1, _F), gn2_ms.reshape(1, _F),
                 mW1, mb1.reshape(1, 64), mW2, mb2.reshape(1, 16), gum)
    return y[:_N]


# speedup vs baseline: 1.2181x; 1.2181x over previous
import functools

import jax
import jax.numpy as jnp
from jax import lax
from jax.experimental import pallas as pl
from jax.experimental.pallas import tpu as pltpu
from jax.experimental.pallas import tpu_sc as plsc

_N = 10000
_F = 128
_NPAD = 10240
_PADIDX = _NPAD - 1
_CHUNK = 128
_NC = 2
_NS = 16
_NW = _NC * _NS
_RPS = _NPAD // _NS
_EPS = 1e-5
_TAU = 0.1


def _leaky(v):
    return jnp.where(v >= 0, v, 0.01 * v)


@functools.lru_cache(maxsize=None)
def _sc_scatter_kernel(cpw: int, with_degrees: bool):
    mesh = plsc.VectorSubcoreMesh(core_axis_name="c", subcore_axis_name="s")
    out_type = [jax.ShapeDtypeStruct((_NC, _NPAD, _F), jnp.float32)]
    if with_degrees:
        out_type += [jax.ShapeDtypeStruct((_NC, _NPAD), jnp.float32)] * 2
    scratch = [
        pltpu.VMEM((4, 2, _CHUNK), jnp.int32),
        pltpu.VMEM((2, _CHUNK, _F), jnp.float32),
        pltpu.VMEM_SHARED((_NPAD, _F), jnp.float32),
        pltpu.SemaphoreType.DMA,
        pltpu.SemaphoreType.DMA,
        pltpu.SemaphoreType.DMA,
        pltpu.SemaphoreType.DMA,
        pltpu.SemaphoreType.DMA,
        pltpu.SemaphoreType.DMA,
        pltpu.SemaphoreType.DMA,
        pltpu.SemaphoreType.DMA,
    ]
    if with_degrees:
        scratch += [
            pltpu.VMEM((_CHUNK,), jnp.float32),
            pltpu.VMEM_SHARED((_NPAD,), jnp.float32),
            pltpu.VMEM_SHARED((_NPAD,), jnp.float32),
        ]

    def body(*refs):
        if with_degrees:
            (src, idx, z2d, z1d,
             out, dout, bout,
             ibuf, rows, acc, si0, si1, si2, si3,
             sga, sgb, ssa, ssb, ones_v, dacc, bacc) = refs
        else:
            (src, idx, z2d,
             out, ibuf, rows, acc, si0, si1, si2, si3,
             sga, sgb, ssa, ssb) = refs
        isem = (si0, si1, si2, si3)
        gsem = (sga, sgb)
        ssem = (ssa, ssb)
        cid = lax.axis_index("c")
        sid = lax.axis_index("s")
        w = cid * _NS + sid
        j0 = w * cpw
        jlast = j0 + cpw - 1
        r0 = sid * _RPS
        pltpu.sync_copy(z2d.at[pl.ds(r0, _RPS)], acc.at[pl.ds(r0, _RPS)])
        if with_degrees:
            pltpu.sync_copy(z1d.at[pl.ds(r0, _RPS)], dacc.at[pl.ds(r0, _RPS)])
            pltpu.sync_copy(z1d.at[pl.ds(r0, _RPS)], bacc.at[pl.ds(r0, _RPS)])
            for i in range(_CHUNK // 16):
                ones_v[pl.ds(i * 16, 16)] = jnp.ones((16,), jnp.float32)
        plsc.subcore_barrier()
        for s in range(4):
            pltpu.async_copy(idx.at[j0 + s], ibuf.at[s], isem[s])
        for b in range(2):
            pltpu.make_async_copy(idx.at[j0 + b], ibuf.at[b], isem[b]).wait()
            pltpu.async_copy(src.at[ibuf.at[b, 0]], rows.at[b], gsem[b])

        nu = cpw // 4

        def step(u, carry):
            for s in range(4):
                b = s % 2
                j = u * 4 + s
                sn = (s + 2) % 4
                pltpu.make_async_copy(
                    src.at[ibuf.at[s, 0]], rows.at[b], gsem[b]).wait()
                pltpu.async_copy(rows.at[b], acc.at[ibuf.at[s, 1]], ssem[b],
                                 add=True)
                if with_degrees:
                    pltpu.sync_copy(ones_v, dacc.at[ibuf.at[s, 0]], add=True)
                    pltpu.sync_copy(ones_v, bacc.at[ibuf.at[s, 1]], add=True)
                pltpu.make_async_copy(
                    rows.at[b], acc.at[ibuf.at[s, 1]], ssem[b]).wait()
                pltpu.async_copy(
                    idx.at[jnp.minimum(j0 + j + 4, jlast)], ibuf.at[s], isem[s])
                pltpu.make_async_copy(
                    idx.at[jnp.minimum(j0 + j + 2, jlast)], ibuf.at[sn],
                    isem[sn]).wait()
                if s < 2:
                    pltpu.async_copy(src.at[ibuf.at[sn, 0]], rows.at[b], gsem[b])
                else:
                    @pl.when(u < nu - 1)
                    def _issue():
                        pltpu.async_copy(src.at[ibuf.at[sn, 0]], rows.at[b],
                                         gsem[b])
            return carry

        lax.fori_loop(0, nu, step, 0)
        for s in (2, 3):
            pltpu.make_async_copy(idx.at[jlast], ibuf.at[s], isem[s]).wait()
        plsc.subcore_barrier()
        pltpu.sync_copy(acc.at[pl.ds(r0, _RPS)], out.at[cid, pl.ds(r0, _RPS)])
        if with_degrees:
            pltpu.sync_copy(dacc.at[pl.ds(r0, _RPS)], dout.at[cid, pl.ds(r0, _RPS)])
            pltpu.sync_copy(bacc.at[pl.ds(r0, _RPS)], bout.at[cid, pl.ds(r0, _RPS)])

    return pl.kernel(
        body,
        out_type=tuple(out_type) if with_degrees else out_type[0],
        mesh=mesh,
        scratch_types=scratch,
    )


def _tc_matmul(xp, W):
    def body(x_ref, w_ref, o_ref):
        o_ref[...] = jnp.dot(x_ref[...], w_ref[...],
                             preferred_element_type=jnp.float32)
    return pl.pallas_call(
        body, out_shape=jax.ShapeDtypeStruct((_NPAD, _F), jnp.float32))(xp, W)


def _tc_combine_first(e_part, dcnt, bcnt):
    def body(e_ref, d_ref, b_ref, eo_ref, dinv_ref, binv_ref):
        mask = lax.broadcasted_iota(jnp.int32, (_NPAD, 1), 0) < _N
        dc = d_ref[0] + d_ref[1]
        bc = b_ref[0] + b_ref[1]
        dinv = jnp.where(mask & (dc > 0), 1.0 / dc, 0.0)
        binv = jnp.where(mask & (bc > 0), 1.0 / bc, 0.0)
        dinv_ref[...] = dinv
        binv_ref[...] = binv
        eo_ref[...] = (e_ref[0] + e_ref[1]) * binv
    return pl.pallas_call(body, out_shape=(
        jax.ShapeDtypeStruct((_NPAD, _F), jnp.float32),
        jax.ShapeDtypeStruct((_NPAD, 1), jnp.float32),
        jax.ShapeDtypeStruct((_NPAD, 1), jnp.float32),
    ))(e_part, dcnt, bcnt)


def _tc_combine(e_part, binv):
    def body(e_ref, bi_ref, eo_ref):
        eo_ref[...] = (e_ref[0] + e_ref[1]) * bi_ref[...]
    return pl.pallas_call(
        body, out_shape=jax.ShapeDtypeStruct((_NPAD, _F), jnp.float32))(e_part, binv)


def _tc_layer(o_part, dinv, bias, gw, gb, gms, W):
    def body(o_ref, di_ref, bi_ref, gw_ref, gb_ref, gms_ref, w_ref, out_ref):
        mask = lax.broadcasted_iota(jnp.int32, (_NPAD, 1), 0) < _N
        h = (o_ref[0] + o_ref[1]) * di_ref[...] + bi_ref[...]
        h = jnp.where(mask, h, 0.0)
        mean = jnp.sum(h, axis=0, keepdims=True) * (1.0 / _N)
        hc = jnp.where(mask, h - mean * gms_ref[...], 0.0)
        var = jnp.sum(hc * hc, axis=0, keepdims=True) * (1.0 / _N)
        g = hc * lax.rsqrt(var + _EPS) * gw_ref[...] + gb_ref[...]
        g = jnp.where(mask, _leaky(g), 0.0)
        out_ref[...] = jnp.dot(g, w_ref[...], preferred_element_type=jnp.float32)
    return pl.pallas_call(
        body, out_shape=jax.ShapeDtypeStruct((_NPAD, _F), jnp.float32))(
            o_part, dinv, bias, gw, gb, gms, W)


def _tc_head(o_part, dinv, bias, gw, gb, gms, mW1, mb1, mW2, mb2, gum):
    def body(o_ref, di_ref, bi_ref, gw_ref, gb_ref, gms_ref,
             w1_ref, b1_ref, w2_ref, b2_ref, g_ref, out_ref):
        mask = lax.broadcasted_iota(jnp.int32, (_NPAD, 1), 0) < _N
        h = (o_ref[0] + o_ref[1]) * di_ref[...] + bi_ref[...]
        h = jnp.where(mask, h, 0.0)
        mean = jnp.sum(h, axis=0, keepdims=True) * (1.0 / _N)
        hc = jnp.where(mask, h - mean * gms_ref[...], 0.0)
        var = jnp.sum(hc * hc, axis=0, keepdims=True) * (1.0 / _N)
        g = hc * lax.rsqrt(var + _EPS) * gw_ref[...] + gb_ref[...]
        g = jnp.where(mask, _leaky(g), 0.0)
        m = jnp.dot(g, w1_ref[...], preferred_element_type=jnp.float32) + b1_ref[...]
        m = jnp.where(mask, m, 0.0)
        mmean = jnp.sum(m, axis=0, keepdims=True) * (1.0 / _N)
        mc = jnp.where(mask, m - mmean, 0.0)
        mvar = jnp.sum(mc * mc, axis=0, keepdims=True) * (1.0 / _N)
        mi = _leaky(mc * lax.rsqrt(mvar + _EPS))
        logits = jnp.dot(mi, w2_ref[...], preferred_element_type=jnp.float32) + b2_ref[...]
        z = (logits + g_ref[...]) * (1.0 / _TAU)
        z = z - jnp.max(z, axis=1, keepdims=True)
        ez = jnp.exp(z)
        out_ref[...] = ez / jnp.sum(ez, axis=1, keepdims=True)
    return pl.pallas_call(
        body, out_shape=jax.ShapeDtypeStruct((_NPAD, 16), jnp.float32))(
            o_part, dinv, bias, gw, gb, gms, mW1, mb1, mW2, mb2, gum)


def kernel(x, inc_idx, W1, b1, gn1_w, gn1_b, gn1_ms, W2, b2, gn2_w, gn2_b,
           gn2_ms, mW1, mb1, mW2, mb2, gumbel):
    f32 = jnp.float32
    row = inc_idx[0].astype(jnp.int32)
    col = inc_idx[1].astype(jnp.int32)
    nnz = row.shape[0]
    chunks = -(-nnz // _CHUNK)
    cpw = -(-chunks // _NW)
    cpw += cpw % 2
    nnzp = cpw * _NW * _CHUNK
    pad = _N + jnp.arange(nnzp - nnz, dtype=jnp.int32) % (_NPAD - _N)
    rowp = jnp.concatenate([row, pad]).reshape(-1, _CHUNK)
    colp = jnp.concatenate([col, pad]).reshape(-1, _CHUNK)
    idx_ne = jnp.stack([rowp, colp], axis=1)
    idx_en = jnp.stack([colp, rowp], axis=1)
    xp = jnp.zeros((_NPAD, _F), f32).at[:_N, :].set(x)
    z2d = jnp.zeros((_NPAD, _F), f32)
    z1d = jnp.zeros((_NPAD,), f32)
    gum = jnp.zeros((_NPAD, 16), f32).at[:_N, :].set(gumbel)

    scat_deg = _sc_scatter_kernel(cpw, True)
    scat = _sc_scatter_kernel(cpw, False)

    x1 = _tc_matmul(xp, W1)
    e_p, dcnt, bcnt = scat_deg(x1, idx_ne, z2d, z1d)
    e_s, dinv, binv = _tc_combine_first(
        e_p, dcnt.reshape(_NC, _NPAD, 1), bcnt.reshape(_NC, _NPAD, 1))
    o_p = scat(e_s, idx_en, z2d)
    x2 = _tc_layer(o_p, dinv, b1.reshape(1, _F), gn1_w.reshape(1, _F),
                   gn1_b.reshape(1, _F), gn1_ms.reshape(1, _F), W2)
    e2_p = scat(x2, idx_ne, z2d)
    e2_s = _tc_combine(e2_p, binv)
    o2_p = scat(e2_s, idx_en, z2d)
    y = _tc_head(o2_p, dinv, b2.reshape(1, _F), gn2_w.reshape(1, _F),
                 gn2_b.reshape(1, _F), gn2_ms.reshape(1, _F),
                 mW1, mb1.reshape(1, 64), mW2, mb2.reshape(1, 16), gum)
    return y[:_N]
